# baseline (speedup 1.0000x reference)
import functools

import jax
import jax.numpy as jnp
from jax import lax
from jax.experimental import pallas as pl
from jax.experimental.pallas import tpu as pltpu
from jax.experimental.pallas import tpu_sc as plsc

N_FIELDS = 26
VOCAB = 100000
EMBED_DIM = 32
BATCH = 16384
FDIM = N_FIELDS * EMBED_DIM

_NC = 2
_NS = 16
_NW = _NC * _NS
_BPW = BATCH // _NW
_CH = 128


def _sc_body(*args):
    fields = args[:N_FIELDS]
    tables = args[N_FIELDS:2 * N_FIELDS]
    out_hbm = args[2 * N_FIELDS]
    scratch = args[2 * N_FIELDS + 1:]
    idx_v, qidx_v = scratch[:2]
    buf = scratch[2]
    isem, gsem, wsem = scratch[3:]
    wid = lax.axis_index("s") * _NC + lax.axis_index("c")
    base = wid * _BPW

    icopies = [
        pltpu.async_copy(fields[f].at[pl.ds(base, _BPW)], idx_v.at[f], isem)
        for f in range(N_FIELDS)
    ]
    for c in icopies:
        c.wait()

    def body(i, _):
        f = i // (_BPW // 16)
        j = i % (_BPW // 16)
        v = idx_v[f, pl.ds(j * 16, 16)]
        qidx_v[f, pl.ds(j * 16, 16)] = lax.shift_right_logical(v, 2)
        return _

    lax.fori_loop(0, N_FIELDS * (_BPW // 16), body, None)

    for f in range(N_FIELDS):
        def chunk(c, _, f=f):
            g = pltpu.async_copy(
                tables[f].at[qidx_v.at[f, pl.ds(c * _CH, _CH)]], buf, gsem
            )
            g.wait()
            w = pltpu.async_copy(
                buf, out_hbm.at[pl.ds((f * _NW + wid) * _CH, _CH)], wsem
            )
            w.wait()
            return _

        lax.fori_loop(0, 4, chunk, None)


_sc = functools.partial(
    pl.kernel,
    mesh=plsc.VectorSubcoreMesh(core_axis_name="c", subcore_axis_name="s"),
    out_type=jax.ShapeDtypeStruct((N_FIELDS * BATCH * EMBED_DIM // 128, 128),
                                  jnp.float32),
    scratch_types=[
        pltpu.VMEM((N_FIELDS, _BPW), jnp.int32),
        pltpu.VMEM((N_FIELDS, _BPW), jnp.int32),
        pltpu.VMEM((_CH, 128), jnp.float32),
        pltpu.SemaphoreType.DMA, pltpu.SemaphoreType.DMA,
        pltpu.SemaphoreType.DMA,
    ],
)(_sc_body)


def kernel(f0, f1, f2, f3, f4, f5, f6, f7, f8, f9, f10, f11, f12, f13, f14,
           f15, f16, f17, f18, f19, f20, f21, f22, f23, f24, f25, table0,
           table1, table2, table3, table4, table5, table6, table7, table8,
           table9, table10, table11, table12, table13, table14, table15,
           table16, table17, table18, table19, table20, table21, table22,
           table23, table24, table25, gamma, beta):
    fields = (f0, f1, f2, f3, f4, f5, f6, f7, f8, f9, f10, f11, f12, f13,
              f14, f15, f16, f17, f18, f19, f20, f21, f22, f23, f24, f25)
    tables = (table0, table1, table2, table3, table4, table5, table6, table7,
              table8, table9, table10, table11, table12, table13, table14,
              table15, table16, table17, table18, table19, table20, table21,
              table22, table23, table24, table25)
    t128 = tuple(t.reshape(VOCAB * EMBED_DIM // 128, 128) for t in tables)
    x2 = _sc(*fields, *t128)
    return x2


# speedup vs baseline: 1.0231x; 1.0231x over previous
import functools

import jax
import jax.numpy as jnp
from jax import lax
from jax.experimental import pallas as pl
from jax.experimental.pallas import tpu as pltpu
from jax.experimental.pallas import tpu_sc as plsc

N_FIELDS = 26
VOCAB = 100000
EMBED_DIM = 32
BATCH = 16384
FDIM = N_FIELDS * EMBED_DIM
EPS = 1e-5

_NC = 2
_NS = 16
_NW = _NC * _NS
_BPW = BATCH // _NW
_FSZ = BATCH * EMBED_DIM

_NBUF = 4


def _sc_gather_body(nf, *args):
    fields = args[:nf]
    tables = args[nf:2 * nf]
    out_hbm = args[2 * nf]
    scratch = args[2 * nf + 1:]
    idx_v = scratch[0]
    rows = scratch[1:1 + _NBUF]
    isem, gsem, wsem = scratch[1 + _NBUF:]
    wid = lax.axis_index("s") * _NC + lax.axis_index("c")
    base = wid * _BPW

    icopies = [
        pltpu.async_copy(fields[f].at[pl.ds(base, _BPW)], idx_v.at[f], isem)
        for f in range(nf)
    ]
    for c in icopies:
        c.wait()

    gathers = [None] * nf
    writes = [None] * nf
    DEPTH = 2
    _CH = _BPW // 4

    def start_gather(f):
        gathers[f] = pltpu.async_copy(
            tables[f].at[idx_v.at[f]], rows[f % _NBUF], gsem
        )

    def start_write(f):
        writes[f] = [
            pltpu.async_copy(
                rows[f % _NBUF].at[pl.ds(c * _CH, _CH), :],
                out_hbm.at[pl.ds((f * _NW + wid) * _CH, _CH),
                           pl.ds(c * EMBED_DIM, EMBED_DIM)],
                wsem,
            )
            for c in range(4)
        ]

    for f in range(DEPTH):
        start_gather(f)
    for f in range(nf):
        if f + DEPTH < nf:
            if f + DEPTH >= _NBUF:
                for w in writes[f + DEPTH - _NBUF]:
                    w.wait()
            start_gather(f + DEPTH)
        gathers[f].wait()
        start_write(f)
    for f in range(nf - _NBUF, nf):
        for w in writes[f]:
            w.wait()


def _make_sc_gather(nf):
    return functools.partial(
        pl.kernel,
        mesh=plsc.VectorSubcoreMesh(core_axis_name="c", subcore_axis_name="s"),
        compiler_params=pltpu.CompilerParams(use_tc_tiling_on_sc=False),
        out_type=jax.ShapeDtypeStruct((nf * _FSZ // 128, 128), jnp.float32),
        scratch_types=[pltpu.VMEM((nf, _BPW), jnp.int32)]
        + [pltpu.VMEM((_BPW, EMBED_DIM), jnp.float32) for _ in range(_NBUF)]
        + [pltpu.SemaphoreType.DMA, pltpu.SemaphoreType.DMA,
           pltpu.SemaphoreType.DMA],
    )(functools.partial(_sc_gather_body, nf))


_NF_A = 13
_NF_B = N_FIELDS - _NF_A
_sc_gather_a = _make_sc_gather(_NF_A)
_sc_gather_b = _make_sc_gather(_NF_B)


_N_BLOCKS = 16
_BB = BATCH // _N_BLOCKS
_PB = _BB * EMBED_DIM // 128
_PF = _FSZ // 128


def _fold4(v):
    return (v[:, 0:32] + v[:, 32:64]) + (v[:, 64:96] + v[:, 96:128])


def _stats_body(*refs):
    xs = refs[:N_FIELDS]
    g_ref, b_ref, ab_ref, acc_ref = refs[N_FIELDS:]
    i = pl.program_id(0)

    @pl.when(i == 0)
    def _():
        acc_ref[...] = jnp.zeros_like(acc_ref)

    for f in range(N_FIELDS):
        xb = xs[f][...]
        acc_ref[2 * f:2 * f + 1, :] += jnp.sum(xb, axis=0, keepdims=True)
        acc_ref[2 * f + 1:2 * f + 2, :] += jnp.sum(xb * xb, axis=0,
                                                   keepdims=True)

    @pl.when(i == _N_BLOCKS - 1)
    def _():
        s = jnp.concatenate(
            [_fold4(acc_ref[2 * f:2 * f + 1, :]) for f in range(N_FIELDS)],
            axis=-1,
        )
        q = jnp.concatenate(
            [_fold4(acc_ref[2 * f + 1:2 * f + 2, :]) for f in range(N_FIELDS)],
            axis=-1,
        )
        mean = s * (1.0 / BATCH)
        var = q * (1.0 / BATCH) - mean * mean
        a = g_ref[...] * lax.rsqrt(var + EPS)
        ab_ref[0:1, :] = a
        ab_ref[1:2, :] = b_ref[...] - mean * a


def _stats(x2, g, b):
    xa, xb = x2
    srcs = [xa] * _NF_A + [xb] * _NF_B
    return pl.pallas_call(
        _stats_body,
        grid=(_N_BLOCKS,),
        in_specs=[
            pl.BlockSpec(
                (_PB, 128),
                lambda i, f=(f if f < _NF_A else f - _NF_A):
                    (f * _N_BLOCKS + i, 0))
            for f in range(N_FIELDS)
        ] + [
            pl.BlockSpec((1, FDIM), lambda i: (0, 0)),
            pl.BlockSpec((1, FDIM), lambda i: (0, 0)),
        ],
        out_specs=pl.BlockSpec((2, FDIM), lambda i: (0, 0)),
        out_shape=jax.ShapeDtypeStruct((2, FDIM), jnp.float32),
        scratch_shapes=[pltpu.VMEM((2 * N_FIELDS, 128), jnp.float32)],
    )(*srcs, g, b)


def _norm_body(*refs):
    xs = refs[:N_FIELDS]
    ab_ref, o_ref = refs[N_FIELDS:]
    pieces = []
    for f in range(N_FIELDS):
        af = ab_ref[0:1, 32 * f:32 * f + 32]
        bf = ab_ref[1:2, 32 * f:32 * f + 32]
        a128 = jnp.concatenate([af, af, af, af], axis=-1)
        b128 = jnp.concatenate([bf, bf, bf, bf], axis=-1)
        y = xs[f][...] * a128 + b128
        piece = jnp.concatenate(
            [y[g * 128:(g + 1) * 128, 32 * c:32 * c + 32]
             for g in range(2) for c in range(4)],
            axis=0,
        )
        pieces.append(piece)
    o_ref[...] = jnp.concatenate(pieces, axis=-1)


def _norm(x2, ab):
    xa, xb = x2
    srcs = [xa] * _NF_A + [xb] * _NF_B
    return pl.pallas_call(
        _norm_body,
        grid=(_N_BLOCKS,),
        in_specs=[
            pl.BlockSpec(
                (_PB, 128),
                lambda i, f=(f if f < _NF_A else f - _NF_A):
                    (f * _N_BLOCKS + i, 0))
            for f in range(N_FIELDS)
        ] + [pl.BlockSpec((2, FDIM), lambda i: (0, 0))],
        out_specs=pl.BlockSpec((_BB, FDIM), lambda i: (i, 0)),
        out_shape=jax.ShapeDtypeStruct((BATCH, FDIM), jnp.float32),
    )(*srcs, ab)


def kernel(f0, f1, f2, f3, f4, f5, f6, f7, f8, f9, f10, f11, f12, f13, f14,
           f15, f16, f17, f18, f19, f20, f21, f22, f23, f24, f25, table0,
           table1, table2, table3, table4, table5, table6, table7, table8,
           table9, table10, table11, table12, table13, table14, table15,
           table16, table17, table18, table19, table20, table21, table22,
           table23, table24, table25, gamma, beta):
    fields = (f0, f1, f2, f3, f4, f5, f6, f7, f8, f9, f10, f11, f12, f13,
              f14, f15, f16, f17, f18, f19, f20, f21, f22, f23, f24, f25)
    tables = (table0, table1, table2, table3, table4, table5, table6, table7,
              table8, table9, table10, table11, table12, table13, table14,
              table15, table16, table17, table18, table19, table20, table21,
              table22, table23, table24, table25)
    xa = _sc_gather_a(*fields[:_NF_A], *tables[:_NF_A])
    xb = _sc_gather_b(*fields[_NF_A:], *tables[_NF_A:])
    ab = _stats((xa, xb), gamma.reshape(1, FDIM), beta.reshape(1, FDIM))
    return _norm((xa, xb), ab)


# speedup vs baseline: 1.0257x; 1.0026x over previous
import functools

import jax
import jax.numpy as jnp
from jax import lax
from jax.experimental import pallas as pl
from jax.experimental.pallas import tpu as pltpu
from jax.experimental.pallas import tpu_sc as plsc

N_FIELDS = 26
VOCAB = 100000
EMBED_DIM = 32
BATCH = 16384
FDIM = N_FIELDS * EMBED_DIM
EPS = 1e-5

_NC = 2
_NS = 16
_NW = _NC * _NS
_BPW = BATCH // _NW
_FSZ = BATCH * EMBED_DIM

_NBUF = 4


def _sc_gather_body(nf, *args):
    fields = args[:nf]
    tables = args[nf:2 * nf]
    out_hbm = args[2 * nf]
    scratch = args[2 * nf + 1:]
    idx_v = scratch[0]
    rows = scratch[1:1 + _NBUF]
    isem, gsem, wsem = scratch[1 + _NBUF:]
    wid = lax.axis_index("s") * _NC + lax.axis_index("c")
    base = wid * _BPW

    icopies = [
        pltpu.async_copy(fields[f].at[pl.ds(base, _BPW)], idx_v.at[f], isem)
        for f in range(nf)
    ]
    for c in icopies:
        c.wait()

    gathers = [None] * nf
    writes = [None] * nf
    DEPTH = 2
    _CH = _BPW // 4

    def start_gather(f):
        gathers[f] = pltpu.async_copy(
            tables[f].at[idx_v.at[f]], rows[f % _NBUF], gsem
        )

    def start_write(f):
        writes[f] = [
            pltpu.async_copy(
                rows[f % _NBUF].at[pl.ds(c * _CH, _CH), :],
                out_hbm.at[pl.ds((f * _NW + wid) * _CH, _CH),
                           pl.ds(c * EMBED_DIM, EMBED_DIM)],
                wsem,
            )
            for c in range(4)
        ]

    for f in range(DEPTH):
        start_gather(f)
    for f in range(nf):
        if f + DEPTH < nf:
            if f + DEPTH >= _NBUF:
                for w in writes[f + DEPTH - _NBUF]:
                    w.wait()
            start_gather(f + DEPTH)
        gathers[f].wait()
        start_write(f)
    for f in range(nf - _NBUF, nf):
        for w in writes[f]:
            w.wait()


def _make_sc_gather(nf):
    return functools.partial(
        pl.kernel,
        mesh=plsc.VectorSubcoreMesh(core_axis_name="c", subcore_axis_name="s"),
        compiler_params=pltpu.CompilerParams(use_tc_tiling_on_sc=False),
        out_type=jax.ShapeDtypeStruct((nf * _FSZ // 128, 128), jnp.float32),
        scratch_types=[pltpu.VMEM((nf, _BPW), jnp.int32)]
        + [pltpu.VMEM((_BPW, EMBED_DIM), jnp.float32) for _ in range(_NBUF)]
        + [pltpu.SemaphoreType.DMA, pltpu.SemaphoreType.DMA,
           pltpu.SemaphoreType.DMA],
    )(functools.partial(_sc_gather_body, nf))


_NF_A = 22
_NF_B = N_FIELDS - _NF_A
_sc_gather_a = _make_sc_gather(_NF_A)
_sc_gather_b = _make_sc_gather(_NF_B)


_N_BLOCKS = 16
_BB = BATCH // _N_BLOCKS
_PB = _BB * EMBED_DIM // 128
_PF = _FSZ // 128


def _fold4(v):
    return (v[:, 0:32] + v[:, 32:64]) + (v[:, 64:96] + v[:, 96:128])


def _stats_body(*refs):
    xs = refs[:N_FIELDS]
    g_ref, b_ref, ab_ref, acc_ref = refs[N_FIELDS:]
    i = pl.program_id(0)

    @pl.when(i == 0)
    def _():
        acc_ref[...] = jnp.zeros_like(acc_ref)

    for f in range(N_FIELDS):
        xb = xs[f][...]
        acc_ref[2 * f:2 * f + 1, :] += jnp.sum(xb, axis=0, keepdims=True)
        acc_ref[2 * f + 1:2 * f + 2, :] += jnp.sum(xb * xb, axis=0,
                                                   keepdims=True)

    @pl.when(i == _N_BLOCKS - 1)
    def _():
        s = jnp.concatenate(
            [_fold4(acc_ref[2 * f:2 * f + 1, :]) for f in range(N_FIELDS)],
            axis=-1,
        )
        q = jnp.concatenate(
            [_fold4(acc_ref[2 * f + 1:2 * f + 2, :]) for f in range(N_FIELDS)],
            axis=-1,
        )
        mean = s * (1.0 / BATCH)
        var = q * (1.0 / BATCH) - mean * mean
        a = g_ref[...] * lax.rsqrt(var + EPS)
        ab_ref[0:1, :] = a
        ab_ref[1:2, :] = b_ref[...] - mean * a


def _stats(x2, g, b):
    xa, xb = x2
    srcs = [xa] * _NF_A + [xb] * _NF_B
    return pl.pallas_call(
        _stats_body,
        grid=(_N_BLOCKS,),
        in_specs=[
            pl.BlockSpec(
                (_PB, 128),
                lambda i, f=(f if f < _NF_A else f - _NF_A):
                    (f * _N_BLOCKS + i, 0))
            for f in range(N_FIELDS)
        ] + [
            pl.BlockSpec((1, FDIM), lambda i: (0, 0)),
            pl.BlockSpec((1, FDIM), lambda i: (0, 0)),
        ],
        out_specs=pl.BlockSpec((2, FDIM), lambda i: (0, 0)),
        out_shape=jax.ShapeDtypeStruct((2, FDIM), jnp.float32),
        scratch_shapes=[pltpu.VMEM((2 * N_FIELDS, 128), jnp.float32)],
    )(*srcs, g, b)


def _norm_body(*refs):
    xs = refs[:N_FIELDS]
    ab_ref, o_ref = refs[N_FIELDS:]
    pieces = []
    for f in range(N_FIELDS):
        af = ab_ref[0:1, 32 * f:32 * f + 32]
        bf = ab_ref[1:2, 32 * f:32 * f + 32]
        a128 = jnp.concatenate([af, af, af, af], axis=-1)
        b128 = jnp.concatenate([bf, bf, bf, bf], axis=-1)
        y = xs[f][...] * a128 + b128
        piece = jnp.concatenate(
            [y[g * 128:(g + 1) * 128, 32 * c:32 * c + 32]
             for g in range(2) for c in range(4)],
            axis=0,
        )
        pieces.append(piece)
    o_ref[...] = jnp.concatenate(pieces, axis=-1)


def _norm(x2, ab):
    xa, xb = x2
    srcs = [xa] * _NF_A + [xb] * _NF_B
    return pl.pallas_call(
        _norm_body,
        grid=(_N_BLOCKS,),
        in_specs=[
            pl.BlockSpec(
                (_PB, 128),
                lambda i, f=(f if f < _NF_A else f - _NF_A):
                    (f * _N_BLOCKS + i, 0))
            for f in range(N_FIELDS)
        ] + [pl.BlockSpec((2, FDIM), lambda i: (0, 0))],
        out_specs=pl.BlockSpec((_BB, FDIM), lambda i: (i, 0)),
        out_shape=jax.ShapeDtypeStruct((BATCH, FDIM), jnp.float32),
    )(*srcs, ab)


def kernel(f0, f1, f2, f3, f4, f5, f6, f7, f8, f9, f10, f11, f12, f13, f14,
           f15, f16, f17, f18, f19, f20, f21, f22, f23, f24, f25, table0,
           table1, table2, table3, table4, table5, table6, table7, table8,
           table9, table10, table11, table12, table13, table14, table15,
           table16, table17, table18, table19, table20, table21, table22,
           table23, table24, table25, gamma, beta):
    fields = (f0, f1, f2, f3, f4, f5, f6, f7, f8, f9, f10, f11, f12, f13,
              f14, f15, f16, f17, f18, f19, f20, f21, f22, f23, f24, f25)
    tables = (table0, table1, table2, table3, table4, table5, table6, table7,
              table8, table9, table10, table11, table12, table13, table14,
              table15, table16, table17, table18, table19, table20, table21,
              table22, table23, table24, table25)
    xa = _sc_gather_a(*fields[:_NF_A], *tables[:_NF_A])
    xb = _sc_gather_b(*fields[_NF_A:], *tables[_NF_A:])
    ab = _stats((xa, xb), gamma.reshape(1, FDIM), beta.reshape(1, FDIM))
    return _norm((xa, xb), ab)


# speedup vs baseline: 1.0317x; 1.0059x over previous
import functools

import jax
import jax.numpy as jnp
from jax import lax
from jax.experimental import pallas as pl
from jax.experimental.pallas import tpu as pltpu
from jax.experimental.pallas import tpu_sc as plsc

N_FIELDS = 26
VOCAB = 100000
EMBED_DIM = 32
BATCH = 16384
FDIM = N_FIELDS * EMBED_DIM
EPS = 1e-5

_NC = 2
_NS = 16
_NW = _NC * _NS
_BPW = BATCH // _NW
_FSZ = BATCH * EMBED_DIM

_NBUF = 4


def _sc_gather_body(nf, *args):
    fields = args[:nf]
    tables = args[nf:2 * nf]
    out_hbm = args[2 * nf]
    scratch = args[2 * nf + 1:]
    idx_v = scratch[0]
    rows = scratch[1:1 + _NBUF]
    isem, gsem, wsem = scratch[1 + _NBUF:]
    wid = lax.axis_index("s") * _NC + lax.axis_index("c")
    base = wid * _BPW

    icopies = [
        pltpu.async_copy(fields[f].at[pl.ds(base, _BPW)], idx_v.at[f], isem)
        for f in range(nf)
    ]
    for c in icopies:
        c.wait()

    gathers = [None] * nf
    writes = [None] * nf
    DEPTH = 2
    _CH = _BPW // 4

    def start_gather(f):
        gathers[f] = pltpu.async_copy(
            tables[f].at[idx_v.at[f]], rows[f % _NBUF], gsem
        )

    def start_write(f):
        writes[f] = [
            pltpu.async_copy(
                rows[f % _NBUF].at[pl.ds(c * _CH, _CH), :],
                out_hbm.at[pl.ds((f * _NW + wid) * _CH, _CH),
                           pl.ds(c * EMBED_DIM, EMBED_DIM)],
                wsem,
            )
            for c in range(4)
        ]

    for f in range(DEPTH):
        start_gather(f)
    for f in range(nf):
        if f + DEPTH < nf:
            if f + DEPTH >= _NBUF:
                for w in writes[f + DEPTH - _NBUF]:
                    w.wait()
            start_gather(f + DEPTH)
        gathers[f].wait()
        start_write(f)
    for f in range(nf - _NBUF, nf):
        for w in writes[f]:
            w.wait()


def _make_sc_gather(nf):
    return functools.partial(
        pl.kernel,
        mesh=plsc.VectorSubcoreMesh(core_axis_name="c", subcore_axis_name="s"),
        compiler_params=pltpu.CompilerParams(use_tc_tiling_on_sc=False),
        out_type=jax.ShapeDtypeStruct((nf * _FSZ // 128, 128), jnp.float32),
        scratch_types=[pltpu.VMEM((nf, _BPW), jnp.int32)]
        + [pltpu.VMEM((_BPW, EMBED_DIM), jnp.float32) for _ in range(_NBUF)]
        + [pltpu.SemaphoreType.DMA, pltpu.SemaphoreType.DMA,
           pltpu.SemaphoreType.DMA],
    )(functools.partial(_sc_gather_body, nf))


_NF_A = 22
_NF_B = N_FIELDS - _NF_A
_sc_gather_a = _make_sc_gather(_NF_A)
_sc_gather_b = _make_sc_gather(_NF_B)


_N_BLOCKS = 8
_BB = BATCH // _N_BLOCKS
_PB = _BB * EMBED_DIM // 128
_PF = _FSZ // 128


def _fold4(v):
    return (v[:, 0:32] + v[:, 32:64]) + (v[:, 64:96] + v[:, 96:128])


def _stats_body(*refs):
    xs = refs[:N_FIELDS]
    g_ref, b_ref, ab_ref, acc_ref = refs[N_FIELDS:]
    i = pl.program_id(0)

    @pl.when(i == 0)
    def _():
        acc_ref[...] = jnp.zeros_like(acc_ref)

    for f in range(N_FIELDS):
        xb = xs[f][...]
        acc_ref[2 * f:2 * f + 1, :] += jnp.sum(xb, axis=0, keepdims=True)
        acc_ref[2 * f + 1:2 * f + 2, :] += jnp.sum(xb * xb, axis=0,
                                                   keepdims=True)

    @pl.when(i == _N_BLOCKS - 1)
    def _():
        s = jnp.concatenate(
            [_fold4(acc_ref[2 * f:2 * f + 1, :]) for f in range(N_FIELDS)],
            axis=-1,
        )
        q = jnp.concatenate(
            [_fold4(acc_ref[2 * f + 1:2 * f + 2, :]) for f in range(N_FIELDS)],
            axis=-1,
        )
        mean = s * (1.0 / BATCH)
        var = q * (1.0 / BATCH) - mean * mean
        a = g_ref[...] * lax.rsqrt(var + EPS)
        ab_ref[0:1, :] = a
        ab_ref[1:2, :] = b_ref[...] - mean * a


def _stats(x2, g, b):
    xa, xb = x2
    srcs = [xa] * _NF_A + [xb] * _NF_B
    return pl.pallas_call(
        _stats_body,
        grid=(_N_BLOCKS,),
        in_specs=[
            pl.BlockSpec(
                (_PB, 128),
                lambda i, f=(f if f < _NF_A else f - _NF_A):
                    (f * _N_BLOCKS + i, 0))
            for f in range(N_FIELDS)
        ] + [
            pl.BlockSpec((1, FDIM), lambda i: (0, 0)),
            pl.BlockSpec((1, FDIM), lambda i: (0, 0)),
        ],
        out_specs=pl.BlockSpec((2, FDIM), lambda i: (0, 0)),
        out_shape=jax.ShapeDtypeStruct((2, FDIM), jnp.float32),
        scratch_shapes=[pltpu.VMEM((2 * N_FIELDS, 128), jnp.float32)],
    )(*srcs, g, b)


def _norm_body(*refs):
    xs = refs[:N_FIELDS]
    ab_ref, o_ref = refs[N_FIELDS:]
    pieces = []
    for f in range(N_FIELDS):
        af = ab_ref[0:1, 32 * f:32 * f + 32]
        bf = ab_ref[1:2, 32 * f:32 * f + 32]
        a128 = jnp.concatenate([af, af, af, af], axis=-1)
        b128 = jnp.concatenate([bf, bf, bf, bf], axis=-1)
        y = xs[f][...] * a128 + b128
        piece = jnp.concatenate(
            [y[g * 128:(g + 1) * 128, 32 * c:32 * c + 32]
             for g in range(_BB // 512) for c in range(4)],
            axis=0,
        )
        pieces.append(piece)
    o_ref[...] = jnp.concatenate(pieces, axis=-1)


def _norm(x2, ab):
    xa, xb = x2
    srcs = [xa] * _NF_A + [xb] * _NF_B
    return pl.pallas_call(
        _norm_body,
        grid=(_N_BLOCKS,),
        in_specs=[
            pl.BlockSpec(
                (_PB, 128),
                lambda i, f=(f if f < _NF_A else f - _NF_A):
                    (f * _N_BLOCKS + i, 0))
            for f in range(N_FIELDS)
        ] + [pl.BlockSpec((2, FDIM), lambda i: (0, 0))],
        out_specs=pl.BlockSpec((_BB, FDIM), lambda i: (i, 0)),
        out_shape=jax.ShapeDtypeStruct((BATCH, FDIM), jnp.float32),
    )(*srcs, ab)


def kernel(f0, f1, f2, f3, f4, f5, f6, f7, f8, f9, f10, f11, f12, f13, f14,
           f15, f16, f17, f18, f19, f20, f21, f22, f23, f24, f25, table0,
           table1, table2, table3, table4, table5, table6, table7, table8,
           table9, table10, table11, table12, table13, table14, table15,
           table16, table17, table18, table19, table20, table21, table22,
           table23, table24, table25, gamma, beta):
    fields = (f0, f1, f2, f3, f4, f5, f6, f7, f8, f9, f10, f11, f12, f13,
              f14, f15, f16, f17, f18, f19, f20, f21, f22, f23, f24, f25)
    tables = (table0, table1, table2, table3, table4, table5, table6, table7,
              table8, table9, table10, table11, table12, table13, table14,
              table15, table16, table17, table18, table19, table20, table21,
              table22, table23, table24, table25)
    xa = _sc_gather_a(*fields[:_NF_A], *tables[:_NF_A])
    xb = _sc_gather_b(*fields[_NF_A:], *tables[_NF_A:])
    ab = _stats((xa, xb), gamma.reshape(1, FDIM), beta.reshape(1, FDIM))
    return _norm((xa, xb), ab)
